# Initial kernel scaffold; baseline (speedup 1.0000x reference)
import jax
import jax.numpy as jnp
from jax.experimental import pallas as pl


def kernel(idx, table):
    raise NotImplementedError("write your pallas kernel here")


# speedup vs baseline: 1.0125x; 1.0125x over previous
import jax
import jax.numpy as jnp
from jax import lax
from jax.experimental import pallas as pl
from jax.experimental.pallas import tpu as pltpu
from jax.experimental.pallas import tpu_sc as plsc

D = 32
B = 16384
H = 50
NC, NS, L = 2, 16, 16
NW = NC * NS
EPW = B // NW
CHUNK = 16
STEPS = EPW // CHUNK
RPC = CHUNK * H
GSEG = 80
NGS = RPC // GSEG
RAWW = CHUNK * H * 2


def _body(idx_hbm, table_hbm, out_hbm, raw_v, idxs_v, rows_v, out_v, gsem):
    wid = lax.axis_index("s") * NC + lax.axis_index("c")
    ebase = wid * EPW

    def step(s, carry):
        e0 = ebase + s * CHUNK
        pltpu.sync_copy(idx_hbm.at[pl.ds(e0 * H * 2, RAWW)], raw_v)
        lanes = lax.iota(jnp.int32, L)
        for g in range(RAWW // (2 * L)):
            w = g * (2 * L) + 2 * lanes
            pred = jnp.maximum(plsc.load_gather(raw_v, [w]), 0)
            q, c = divmod(g * L, GSEG)
            idxs_v[q, pl.ds(c, L)] = pred
        cps = [
            pltpu.async_copy(
                table_hbm.at[idxs_v.at[q]],
                rows_v.at[pl.ds(q * GSEG, GSEG)],
                gsem,
            )
            for q in range(NGS)
        ]
        for cp in cps:
            cp.wait()
        scale = jnp.float32(1.0 / H)

        def pool(e, c2):
            r0 = e * H
            acc0 = jnp.zeros((L,), jnp.float32)
            acc1 = jnp.zeros((L,), jnp.float32)
            for j in range(H):
                acc0 = acc0 + rows_v[r0 + j, pl.ds(0, L)]
                acc1 = acc1 + rows_v[r0 + j, pl.ds(L, L)]
            out_v[e, pl.ds(0, L)] = acc0 * scale
            out_v[e, pl.ds(L, L)] = acc1 * scale
            return c2

        lax.fori_loop(0, CHUNK, pool, 0)
        pltpu.sync_copy(out_v, out_hbm.at[pl.ds(e0, CHUNK)])
        return carry

    lax.fori_loop(0, STEPS, step, 0)


def kernel(idx, table):
    idx_flat = idx.reshape(B * H * 2)
    k = pl.kernel(
        _body,
        out_type=jax.ShapeDtypeStruct((B, D), jnp.float32),
        mesh=plsc.VectorSubcoreMesh(core_axis_name="c", subcore_axis_name="s"),
        scratch_types=[
            pltpu.VMEM((RAWW,), jnp.int32),
            pltpu.VMEM((NGS, GSEG), jnp.int32),
            pltpu.VMEM((RPC, D), jnp.float32),
            pltpu.VMEM((CHUNK, D), jnp.float32),
            pltpu.SemaphoreType.DMA,
        ],
        compiler_params=pltpu.CompilerParams(
            needs_layout_passes=False, use_tc_tiling_on_sc=False
        ),
    )
    return k(idx_flat, table)


# speedup vs baseline: 2.6785x; 2.6455x over previous
import jax
import jax.numpy as jnp
from jax import lax
from jax.experimental import pallas as pl
from jax.experimental.pallas import tpu as pltpu
from jax.experimental.pallas import tpu_sc as plsc

D = 32
B = 16384
H = 50
NC, NS, L = 2, 16, 16
NW = NC * NS
EPW = B // NW
CHUNK = 16
STEPS = EPW // CHUNK
RPC = CHUNK * H
GSEG = 80
NGS = RPC // GSEG


def _body(pred_hbm, table_hbm, out_hbm, raw_v, idxs_v, rows_v, out_v, gsem):
    wid = lax.axis_index("s") * NC + lax.axis_index("c")
    ebase = wid * EPW

    def step(s, carry):
        e0 = ebase + s * CHUNK
        pltpu.sync_copy(pred_hbm.at[pl.ds(e0 * H, RPC)], raw_v)
        for g in range(RPC // L):
            v = jnp.maximum(raw_v[pl.ds(g * L, L)], 0)
            q, c = divmod(g * L, GSEG)
            idxs_v[q, pl.ds(c, L)] = v
        cps = [
            pltpu.async_copy(
                table_hbm.at[idxs_v.at[q]],
                rows_v.at[pl.ds(q * GSEG, GSEG)],
                gsem,
            )
            for q in range(NGS)
        ]
        for cp in cps:
            cp.wait()
        scale = jnp.float32(1.0 / H)

        def pool(e, c2):
            r0 = e * H
            acc0 = jnp.zeros((L,), jnp.float32)
            acc1 = jnp.zeros((L,), jnp.float32)
            for j in range(H):
                acc0 = acc0 + rows_v[r0 + j, pl.ds(0, L)]
                acc1 = acc1 + rows_v[r0 + j, pl.ds(L, L)]
            out_v[e, pl.ds(0, L)] = acc0 * scale
            out_v[e, pl.ds(L, L)] = acc1 * scale
            return c2

        lax.fori_loop(0, CHUNK, pool, 0)
        pltpu.sync_copy(out_v, out_hbm.at[pl.ds(e0, CHUNK)])
        return carry

    lax.fori_loop(0, STEPS, step, 0)


def kernel(idx, table):
    pred = idx[:, :, 0].reshape(B * H)
    k = pl.kernel(
        _body,
        out_type=jax.ShapeDtypeStruct((B, D), jnp.float32),
        mesh=plsc.VectorSubcoreMesh(core_axis_name="c", subcore_axis_name="s"),
        scratch_types=[
            pltpu.VMEM((RPC,), jnp.int32),
            pltpu.VMEM((NGS, GSEG), jnp.int32),
            pltpu.VMEM((RPC, D), jnp.float32),
            pltpu.VMEM((CHUNK, D), jnp.float32),
            pltpu.SemaphoreType.DMA,
        ],
        compiler_params=pltpu.CompilerParams(
            needs_layout_passes=False, use_tc_tiling_on_sc=False
        ),
    )
    return k(pred, table)


# speedup vs baseline: 3.0413x; 1.1355x over previous
import jax
import jax.numpy as jnp
from jax import lax
from jax.experimental import pallas as pl
from jax.experimental.pallas import tpu as pltpu
from jax.experimental.pallas import tpu_sc as plsc

D = 32
B = 16384
H = 50
NC, NS, L = 2, 16, 16
NW = NC * NS
EPW = B // NW
CHUNK = 128
STEPS = EPW // CHUNK
NIB = 3
VL = CHUNK // L


def _body(pred_hbm, table_hbm, out_hbm, idxs_v, acc_v, psem, gsem0, gsem1,
          osem0, osem1):
    wid = lax.axis_index("s") * NC + lax.axis_index("c")
    ebase = wid * EPW
    gsems = [gsem0, gsem1]
    osems = [osem0, osem1]
    scale = jnp.float32(1.0 / H)
    zv = jnp.zeros((L,), jnp.float32)

    def stage(s):
        ib = s % NIB
        e0 = ebase + s * CHUNK
        return pltpu.async_copy(
            pred_hbm.at[:, pl.ds(e0, CHUNK)], idxs_v.at[ib], psem
        )

    def clamp(s):
        ib = s % NIB

        def row(j, c):
            for k in range(VL):
                sl = pl.ds(k * L, L)
                idxs_v[ib, j, sl] = jnp.maximum(idxs_v[ib, j, sl], 0)
            return c

        lax.fori_loop(0, H, row, 0)

    def zero(s):
        b = s & 1

        def row(e, c):
            acc_v[b, e, pl.ds(0, L)] = zv
            acc_v[b, e, pl.ds(L, L)] = zv
            return c

        lax.fori_loop(0, CHUNK, row, 0)

    def fire(s):
        ib, b = s % NIB, s & 1

        def one(j, c):
            pltpu.async_copy(
                table_hbm.at[idxs_v.at[ib, j]], acc_v.at[b], gsems[b],
                add=True,
            )
            return c

        lax.fori_loop(0, H, one, 0)

    def drain(s):
        ib, b = s % NIB, s & 1

        def one(j, c):
            pltpu.make_async_copy(
                table_hbm.at[idxs_v.at[ib, j]], acc_v.at[b], gsems[b]
            ).wait()
            return c

        lax.fori_loop(0, H, one, 0)

    def scale_rows(s):
        b = s & 1

        def row(e, c):
            acc_v[b, e, pl.ds(0, L)] = acc_v[b, e, pl.ds(0, L)] * scale
            acc_v[b, e, pl.ds(L, L)] = acc_v[b, e, pl.ds(L, L)] * scale
            return c

        lax.fori_loop(0, CHUNK, row, 0)

    def write(s):
        b = s & 1
        e0 = ebase + s * CHUNK
        return pltpu.async_copy(acc_v.at[b], out_hbm.at[pl.ds(e0, CHUNK)],
                                osems[b])

    pcp = [None] * STEPS
    ocp = [None] * STEPS
    pcp[0] = stage(0)
    for s in range(STEPS):
        b = s & 1
        pcp[s].wait()
        clamp(s)
        if s + 1 < STEPS:
            pcp[s + 1] = stage(s + 1)
        if s >= 2:
            ocp[s - 2].wait()
        zero(s)
        fire(s)
        if s >= 1:
            drain(s - 1)
            scale_rows(s - 1)
            ocp[s - 1] = write(s - 1)
    drain(STEPS - 1)
    scale_rows(STEPS - 1)
    ocp[STEPS - 1] = write(STEPS - 1)
    ocp[STEPS - 2].wait()
    ocp[STEPS - 1].wait()


def kernel(idx, table):
    pred = idx[:, :, 0].T
    k = pl.kernel(
        _body,
        out_type=jax.ShapeDtypeStruct((B, D), jnp.float32),
        mesh=plsc.VectorSubcoreMesh(core_axis_name="c", subcore_axis_name="s"),
        scratch_types=[
            pltpu.VMEM((NIB, H, CHUNK), jnp.int32),
            pltpu.VMEM((2, CHUNK, D), jnp.float32),
            pltpu.SemaphoreType.DMA,
            pltpu.SemaphoreType.DMA,
            pltpu.SemaphoreType.DMA,
            pltpu.SemaphoreType.DMA,
            pltpu.SemaphoreType.DMA,
        ],
        compiler_params=pltpu.CompilerParams(
            needs_layout_passes=False, use_tc_tiling_on_sc=False
        ),
    )
    return k(pred, table)
